# baseline (speedup 1.0000x reference)
import jax
import jax.numpy as jnp
from jax import lax
from jax.experimental import pallas as pl
from jax.experimental.pallas import tpu as pltpu

_NE = 8192
_D = 32
_B = 16
_T = 1024
_N = _B * _T
_TM = 512
_NBLK = _N // _TM
_CN = 2048
_NCHUNK = _NE // _CN
_CCOST = 0.25


def _vq_body(zt_ref, w_ref, zsq_ref, wsq_ref, zq_ref, codes_ref, loss_ref):
    blk = pl.program_id(0)
    zt = zt_ref[...]
    ztr = zt.astype(jnp.bfloat16).astype(jnp.float32)
    zsq = zsq_ref[...]

    minval = jnp.full((1, _TM), jnp.inf, jnp.float32)
    minidx = jnp.zeros((1, _TM), jnp.int32)
    for i in range(_NCHUNK):
        wc = w_ref[i * _CN:(i + 1) * _CN, :]
        s = lax.dot_general(wc, ztr, (((1,), (0,)), ((), ())),
                            preferred_element_type=jnp.float32)
        wsq = wsq_ref[i * _CN:(i + 1) * _CN, :]
        d = (zsq - 2.0 * s) + wsq
        cmin = jnp.min(d, axis=0, keepdims=True)
        rows = lax.broadcasted_iota(jnp.int32, (_CN, _TM), 0) + i * _CN
        cand = jnp.where(d == cmin, rows, _NE)
        cidx = jnp.min(cand, axis=0, keepdims=True)
        take = cmin < minval
        minval = jnp.where(take, cmin, minval)
        minidx = jnp.where(take, cidx, minidx)

    q = jnp.zeros((_D, _TM), jnp.float32)
    for i in range(_NCHUNK):
        wc = w_ref[i * _CN:(i + 1) * _CN, :]
        hi = wc.astype(jnp.bfloat16)
        r1 = wc - hi.astype(jnp.float32)
        mid = r1.astype(jnp.bfloat16)
        lo = (r1 - mid.astype(jnp.float32)).astype(jnp.bfloat16)
        rows = lax.broadcasted_iota(jnp.int32, (_CN, _TM), 0) + i * _CN
        oh = (rows == minidx).astype(jnp.bfloat16)
        dn = (((0,), (0,)), ((), ()))
        q = q + lax.dot_general(hi, oh, dn, preferred_element_type=jnp.float32)
        q = q + lax.dot_general(mid, oh, dn, preferred_element_type=jnp.float32)
        q = q + lax.dot_general(lo, oh, dn, preferred_element_type=jnp.float32)
    zq_ref[...] = q
    codes_ref[...] = minidx
    diff = q - zt
    part = jnp.sum(diff * diff) * (_CCOST / (_N * _D))

    @pl.when(blk == 0)
    def _():
        loss_ref[0, 0] = 0.0

    loss_ref[0, 0] += part


@jax.jit
def kernel(z, W):
    zf = jnp.transpose(z, (0, 2, 1)).reshape(_N, _D)
    zsq_col = lax.optimization_barrier(
        jnp.sum(zf ** 2, axis=1, keepdims=True))
    zsq = jnp.transpose(zsq_col, (1, 0))
    zT = jnp.transpose(z, (1, 0, 2)).reshape(_D, _N)
    wsq_row = lax.optimization_barrier(
        jnp.sum(W ** 2, axis=1))
    wsq = wsq_row.reshape(_NE, 1)
    zq_t, codes, loss = pl.pallas_call(
        _vq_body,
        grid=(_NBLK,),
        in_specs=[
            pl.BlockSpec((_D, _TM), lambda b: (0, b)),
            pl.BlockSpec((_NE, _D), lambda b: (0, 0)),
            pl.BlockSpec((1, _TM), lambda b: (0, b)),
            pl.BlockSpec((_NE, 1), lambda b: (0, 0)),
        ],
        out_specs=[
            pl.BlockSpec((_D, _TM), lambda b: (0, b)),
            pl.BlockSpec((1, _TM), lambda b: (0, b)),
            pl.BlockSpec((1, 1), lambda b: (0, 0), memory_space=pltpu.SMEM),
        ],
        out_shape=[
            jax.ShapeDtypeStruct((_D, _N), jnp.float32),
            jax.ShapeDtypeStruct((1, _N), jnp.int32),
            jax.ShapeDtypeStruct((1, 1), jnp.float32),
        ],
    )(zT, W, zsq, wsq)
    zq = jnp.transpose(zq_t.reshape(_D, _B, _T), (1, 0, 2))
    return zq, loss[0, 0], codes.reshape(_B, _T)
